# baseline (speedup 1.0000x reference)
import jax
import jax.numpy as jnp
from jax.experimental import pallas as pl
from jax.experimental.pallas import tpu as pltpu

_BM = 400


def _gcn_kernel(x_ref, w_ref, adj_ref, out_ref, support_ref):
    @pl.when(pl.program_id(0) == 0)
    def _():
        support_ref[...] = jax.lax.dot(
            x_ref[...].astype(jnp.bfloat16),
            w_ref[...].astype(jnp.bfloat16),
            preferred_element_type=jnp.float32,
        ).astype(jnp.bfloat16)

    out_ref[...] = jax.lax.dot(
        adj_ref[...].astype(jnp.bfloat16),
        support_ref[...],
        preferred_element_type=jnp.float32,
    )


def kernel(input, adj, W):
    n, d_in = input.shape
    d_out = W.shape[1]
    grid = (n // _BM,)
    return pl.pallas_call(
        _gcn_kernel,
        grid=grid,
        in_specs=[
            pl.BlockSpec((n, d_in), lambda i: (0, 0)),
            pl.BlockSpec((d_in, d_out), lambda i: (0, 0)),
            pl.BlockSpec((_BM, n), lambda i: (i, 0)),
        ],
        out_specs=pl.BlockSpec((_BM, d_out), lambda i: (i, 0)),
        out_shape=jax.ShapeDtypeStruct((n, d_out), jnp.float32),
        scratch_shapes=[pltpu.VMEM((n, d_out), jnp.bfloat16)],
    )(input, W, adj)
